# baseline (device time: 769838 ns/iter reference)
import jax
import jax.numpy as jnp
from jax import lax
from jax.experimental import pallas as pl
from jax.experimental.pallas import tpu as pltpu

N_DEV = 4
B = 16
NB_X = 3
NQ = 4

P1R = list(range(0, 4))
P1L = list(range(4, 8))
P2_A, P2_B, P2_C, P2_D = 8, 9, 10, 11


def kernel(x):
    m, n = x.shape
    q = m // NQ
    r = m // B
    bq = B // NQ

    def body(x_ref, recv16, out_ref, own16, xstage, cstage, ostage,
             xin_sems, xout_sems, cin_sems, cout_sems, send_sems, recv_sems):
        my = lax.axis_index("i")
        left = (my - 1) % N_DEV
        right = (my + 1) % N_DEV

        barrier_sem = pltpu.get_barrier_semaphore()
        for nbr in (left, right):
            pl.semaphore_signal(
                barrier_sem, inc=1,
                device_id=(nbr,), device_id_type=pl.DeviceIdType.MESH,
            )
        pl.semaphore_wait(barrier_sem, 2)

        def rdma(src, dst, sem, dev):
            return pltpu.make_async_remote_copy(
                src_ref=src, dst_ref=dst,
                send_sem=send_sems.at[sem], recv_sem=recv_sems.at[sem],
                device_id=(dev,), device_id_type=pl.DeviceIdType.MESH,
            )

        p1r = [rdma(own16.at[pl.ds(k * q, q)],
                    recv16.at[0, pl.ds(k * q, q)], P1R[k], right)
               for k in range(NQ)]
        p1l = [rdma(own16.at[pl.ds(k * q, q)],
                    recv16.at[1, pl.ds(k * q, q)], P1L[k], left)
               for k in range(NQ)]
        p2 = {
            P2_A: rdma(recv16.at[0, pl.ds(0, q)],
                       recv16.at[2, pl.ds(0, q)], P2_A, right),
            P2_B: rdma(recv16.at[0, pl.ds(q, q)],
                       recv16.at[2, pl.ds(q, q)], P2_B, right),
            P2_C: rdma(recv16.at[1, pl.ds(2 * q, q)],
                       recv16.at[2, pl.ds(2 * q, q)], P2_C, left),
            P2_D: rdma(recv16.at[1, pl.ds(3 * q, q)],
                       recv16.at[2, pl.ds(3 * q, q)], P2_D, left),
        }

        def x_in(b, s):
            return pltpu.make_async_copy(
                x_ref.at[pl.ds(b * r, r)], xstage.at[s], xin_sems.at[s])

        for s in range(NB_X - 1):
            x_in(s, s).start()
        for b in range(B):
            s = b % NB_X
            x_in(b, s).wait()
            own16[b * r:(b + 1) * r, :] = xstage[s].astype(jnp.bfloat16)
            if b + NB_X - 1 < B:
                x_in(b + NB_X - 1, (b + NB_X - 1) % NB_X).start()
            if (b + 1) % bq == 0:
                k = (b + 1) // bq - 1
                p1r[k].start()
                p1l[k].start()

        def o_out(b, s, origin_row):
            return pltpu.make_async_copy(
                ostage.at[s], out_ref.at[pl.ds(origin_row + b * r, r)],
                cout_sems.at[s])

        my_row = my * m
        for b in range(B):
            s = b % 2
            if b >= 2:
                o_out(b - 2, s, my_row).wait()
            ostage[s] = own16[b * r:(b + 1) * r, :].astype(jnp.float32)
            o_out(b, s, my_row).start()
        o_out(B - 2, B % 2, my_row).wait()
        o_out(B - 1, (B - 1) % 2, my_row).wait()

        def convert_rows(slot, b0, nb, origin_row):
            def c_in(b, s):
                return pltpu.make_async_copy(
                    recv16.at[slot, pl.ds((b0 + b) * r, r)], cstage.at[s],
                    cin_sems.at[s])

            c_in(0, 0).start()
            for b in range(nb):
                s = b % 2
                if b + 1 < nb:
                    c_in(b + 1, (b + 1) % 2).start()
                c_in(b, s).wait()
                if b >= 2:
                    o_out(b0 + b - 2, s, origin_row).wait()
                ostage[s] = cstage[s].astype(jnp.float32)
                o_out(b0 + b, s, origin_row).start()
            if nb >= 2:
                o_out(b0 + nb - 2, nb % 2, origin_row).wait()
            o_out(b0 + nb - 1, (nb - 1) % 2, origin_row).wait()

        p1r[0].wait_recv()
        p2[P2_A].start()
        p1r[1].wait_recv()
        p2[P2_B].start()
        convert_rows(0, 0, 2 * bq, left * m)
        p1l[0].wait_recv()
        p1l[1].wait_recv()
        convert_rows(1, 0, 2 * bq, right * m)
        p1l[2].wait_recv()
        p2[P2_C].start()
        p1l[3].wait_recv()
        p2[P2_D].start()
        p1r[2].wait_recv()
        p1r[3].wait_recv()
        convert_rows(0, 2 * bq, 2 * bq, left * m)
        convert_rows(1, 2 * bq, 2 * bq, right * m)

        opp_row = ((my + 2) % N_DEV) * m
        for sem, k in ((P2_A, 0), (P2_C, 2), (P2_B, 1), (P2_D, 3)):
            p2[sem].wait_recv()
            convert_rows(2, k * bq, bq, opp_row)

        for d in p1r + p1l:
            d.wait_send()
        for d in p2.values():
            d.wait_send()

    import numpy as np
    recv_buf = np.zeros((3, m, n), dtype=jnp.bfloat16)
    out = pl.pallas_call(
        body,
        out_shape=jax.ShapeDtypeStruct((N_DEV * m, n), x.dtype),
        in_specs=[
            pl.BlockSpec(memory_space=pl.ANY),
            pl.BlockSpec(memory_space=pl.ANY),
        ],
        out_specs=pl.BlockSpec(memory_space=pl.ANY),
        scratch_shapes=[
            pltpu.VMEM((m, n), jnp.bfloat16),
            pltpu.VMEM((NB_X, r, n), jnp.float32),
            pltpu.VMEM((2, r, n), jnp.bfloat16),
            pltpu.VMEM((2, r, n), jnp.float32),
            pltpu.SemaphoreType.DMA((NB_X,)),
            pltpu.SemaphoreType.DMA((NB_X,)),
            pltpu.SemaphoreType.DMA((2,)),
            pltpu.SemaphoreType.DMA((2,)),
            pltpu.SemaphoreType.DMA((12,)),
            pltpu.SemaphoreType.DMA((12,)),
        ],
        compiler_params=pltpu.CompilerParams(
            collective_id=0,
            vmem_limit_bytes=100 * 1024 * 1024,
        ),
    )(x, recv_buf)
    return out


# device time: 737719 ns/iter; 1.0435x vs baseline; 1.0435x over previous
import jax
import jax.numpy as jnp
from jax import lax
from jax.experimental import pallas as pl
from jax.experimental.pallas import tpu as pltpu

N_DEV = 4
B = 16
NB_X = 3
NQ = 4

P1R = list(range(0, 4))
P1L = list(range(4, 8))
P2_A, P2_B, P2_C, P2_D = 8, 9, 10, 11


def kernel(x):
    m, n = x.shape
    q = m // NQ
    r = m // B
    bq = B // NQ

    def body(x_ref, out_ref, recv16, own16, xstage, cstage, ostage,
             xin_sems, xout_sems, cin_sems, cout_sems, send_sems, recv_sems):
        my = lax.axis_index("i")
        left = (my - 1) % N_DEV
        right = (my + 1) % N_DEV

        barrier_sem = pltpu.get_barrier_semaphore()
        for nbr in (left, right):
            pl.semaphore_signal(
                barrier_sem, inc=1,
                device_id=(nbr,), device_id_type=pl.DeviceIdType.MESH,
            )
        pl.semaphore_wait(barrier_sem, 2)

        def rdma(src, dst, sem, dev):
            return pltpu.make_async_remote_copy(
                src_ref=src, dst_ref=dst,
                send_sem=send_sems.at[sem], recv_sem=recv_sems.at[sem],
                device_id=(dev,), device_id_type=pl.DeviceIdType.MESH,
            )

        p1r = [rdma(own16.at[pl.ds(k * q, q)],
                    recv16.at[0, pl.ds(k * q, q)], P1R[k], right)
               for k in range(NQ)]
        p1l = [rdma(own16.at[pl.ds(k * q, q)],
                    recv16.at[1, pl.ds(k * q, q)], P1L[k], left)
               for k in range(NQ)]
        p2 = {
            P2_A: rdma(recv16.at[0, pl.ds(0, q)],
                       recv16.at[2, pl.ds(0, q)], P2_A, right),
            P2_B: rdma(recv16.at[0, pl.ds(q, q)],
                       recv16.at[2, pl.ds(q, q)], P2_B, right),
            P2_C: rdma(recv16.at[1, pl.ds(2 * q, q)],
                       recv16.at[2, pl.ds(2 * q, q)], P2_C, left),
            P2_D: rdma(recv16.at[1, pl.ds(3 * q, q)],
                       recv16.at[2, pl.ds(3 * q, q)], P2_D, left),
        }

        def x_in(b, s):
            return pltpu.make_async_copy(
                x_ref.at[pl.ds(b * r, r)], xstage.at[s], xin_sems.at[s])

        for s in range(NB_X - 1):
            x_in(s, s).start()
        for b in range(B):
            s = b % NB_X
            x_in(b, s).wait()
            own16[b * r:(b + 1) * r, :] = xstage[s].astype(jnp.bfloat16)
            if b + NB_X - 1 < B:
                x_in(b + NB_X - 1, (b + NB_X - 1) % NB_X).start()
            if (b + 1) % bq == 0:
                k = (b + 1) // bq - 1
                p1r[k].start()
                p1l[k].start()

        def o_out(b, s, origin_row):
            return pltpu.make_async_copy(
                ostage.at[s], out_ref.at[pl.ds(origin_row + b * r, r)],
                cout_sems.at[s])

        my_row = my * m
        for b in range(B):
            s = b % 2
            if b >= 2:
                o_out(b - 2, s, my_row).wait()
            ostage[s] = own16[b * r:(b + 1) * r, :].astype(jnp.float32)
            o_out(b, s, my_row).start()
        o_out(B - 2, B % 2, my_row).wait()
        o_out(B - 1, (B - 1) % 2, my_row).wait()

        def convert_rows(slot, b0, nb, origin_row):
            def c_in(b, s):
                return pltpu.make_async_copy(
                    recv16.at[slot, pl.ds((b0 + b) * r, r)], cstage.at[s],
                    cin_sems.at[s])

            c_in(0, 0).start()
            for b in range(nb):
                s = b % 2
                if b + 1 < nb:
                    c_in(b + 1, (b + 1) % 2).start()
                c_in(b, s).wait()
                if b >= 2:
                    o_out(b0 + b - 2, s, origin_row).wait()
                ostage[s] = cstage[s].astype(jnp.float32)
                o_out(b0 + b, s, origin_row).start()
            if nb >= 2:
                o_out(b0 + nb - 2, nb % 2, origin_row).wait()
            o_out(b0 + nb - 1, (nb - 1) % 2, origin_row).wait()

        p1r[0].wait_recv()
        p2[P2_A].start()
        p1r[1].wait_recv()
        p2[P2_B].start()
        convert_rows(0, 0, 2 * bq, left * m)
        p1l[0].wait_recv()
        p1l[1].wait_recv()
        convert_rows(1, 0, 2 * bq, right * m)
        p1l[2].wait_recv()
        p2[P2_C].start()
        p1l[3].wait_recv()
        p2[P2_D].start()
        p1r[2].wait_recv()
        p1r[3].wait_recv()
        convert_rows(0, 2 * bq, 2 * bq, left * m)
        convert_rows(1, 2 * bq, 2 * bq, right * m)

        opp_row = ((my + 2) % N_DEV) * m
        for sem, k in ((P2_A, 0), (P2_C, 2), (P2_B, 1), (P2_D, 3)):
            p2[sem].wait_recv()
            convert_rows(2, k * bq, bq, opp_row)

        for d in p1r + p1l:
            d.wait_send()
        for d in p2.values():
            d.wait_send()

    out, _ = pl.pallas_call(
        body,
        out_shape=(
            jax.ShapeDtypeStruct((N_DEV * m, n), x.dtype),
            jax.ShapeDtypeStruct((3, m, n), jnp.bfloat16),
        ),
        in_specs=[pl.BlockSpec(memory_space=pl.ANY)],
        out_specs=(
            pl.BlockSpec(memory_space=pl.ANY),
            pl.BlockSpec(memory_space=pl.ANY),
        ),
        scratch_shapes=[
            pltpu.VMEM((m, n), jnp.bfloat16),
            pltpu.VMEM((NB_X, r, n), jnp.float32),
            pltpu.VMEM((2, r, n), jnp.bfloat16),
            pltpu.VMEM((2, r, n), jnp.float32),
            pltpu.SemaphoreType.DMA((NB_X,)),
            pltpu.SemaphoreType.DMA((NB_X,)),
            pltpu.SemaphoreType.DMA((2,)),
            pltpu.SemaphoreType.DMA((2,)),
            pltpu.SemaphoreType.DMA((12,)),
            pltpu.SemaphoreType.DMA((12,)),
        ],
        compiler_params=pltpu.CompilerParams(
            collective_id=0,
            vmem_limit_bytes=100 * 1024 * 1024,
        ),
    )(x)
    return out
